# Initial kernel scaffold; baseline (speedup 1.0000x reference)
import functools

import numpy as np
import jax
import jax.numpy as jnp
from jax import lax
from jax.experimental import pallas as pl
from jax.experimental.pallas import tpu as pltpu
from jax.experimental.pallas import tpu_sc as plsc

B = 16384
F = 26
V = 100000
D = 32
NUM = 13
H = 128

NW = 32
BPW = B // NW
IPW = BPW * F
CH = 128
NCH = IPW // CH
G = 4
NG = NCH // G

_OFF_NP = ((np.arange(B * F, dtype=np.int64) % F) * V).astype(np.int32)
_OFF_NP = _OFF_NP.reshape(NW, NCH, CH)


def _sc_gather(cat3, off3, tab2):
    mesh = plsc.VectorSubcoreMesh(core_axis_name="c", subcore_axis_name="s")

    @functools.partial(
        pl.kernel,
        mesh=mesh,
        out_type=jax.ShapeDtypeStruct((NW, NG, G, CH, D), jnp.float32),
        scratch_types=[
            pltpu.VMEM((NCH, CH), jnp.int32),
            pltpu.VMEM((NCH, CH), jnp.int32),
            pltpu.VMEM((G, CH, D), jnp.float32),
            pltpu.VMEM((G, CH, D), jnp.float32),
            pltpu.SemaphoreType.DMA,
            pltpu.SemaphoreType.DMA,
        ],
    )
    def k(cat_h, off_h, tab_h, out_h, idx_v, off_v, buf0, buf1, sem0, sem1):
        wid = lax.axis_index("s") * 2 + lax.axis_index("c")
        pltpu.sync_copy(cat_h.at[wid], idx_v)
        pltpu.sync_copy(off_h.at[wid], off_v)

        def add_body(i, carry):
            r = i // (CH // 16)
            c = (i % (CH // 16)) * 16
            idx_v[r, pl.ds(c, 16)] = (
                idx_v[r, pl.ds(c, 16)] + off_v[r, pl.ds(c, 16)]
            )
            return carry

        lax.fori_loop(0, NCH * (CH // 16), add_body, 0)

        def fire(g, buf, sem):
            return [
                pltpu.async_copy(tab_h.at[idx_v.at[g * G + j]], buf.at[j], sem)
                for j in range(G)
            ]

        def group_body(i, carry):
            g0 = i * 2
            g1 = g0 + 1
            cps0 = fire(g0, buf0, sem0)
            cps1 = fire(g1, buf1, sem1)
            for cp in cps0:
                cp.wait()
            pltpu.sync_copy(buf0, out_h.at[wid, g0])
            for cp in cps1:
                cp.wait()
            pltpu.sync_copy(buf1, out_h.at[wid, g1])
            return carry

        lax.fori_loop(0, NG // 2, group_body, 0)

    return k(cat3, off3, tab2)


def _mlp(embs, num, w1a, w1b, b1r, w2r, b2r):
    BLK = 1024

    def body(e_ref, n_ref, w1a_ref, w1b_ref, b1_ref, w2_ref, b2_ref, o_ref):
        x = jnp.dot(e_ref[...], w1a_ref[...], preferred_element_type=jnp.float32)
        x = x + jnp.dot(n_ref[...], w1b_ref[...], preferred_element_type=jnp.float32)
        x = jnp.maximum(x + b1_ref[...], 0.0)
        y = jnp.sum(x * w2_ref[...], axis=1, keepdims=True) + b2_ref[...]
        o_ref[...] = jax.nn.sigmoid(y)

    return pl.pallas_call(
        body,
        grid=(B // BLK,),
        in_specs=[
            pl.BlockSpec((BLK, F * D), lambda i: (i, 0)),
            pl.BlockSpec((BLK, NUM), lambda i: (i, 0)),
            pl.BlockSpec((F * D, H), lambda i: (0, 0)),
            pl.BlockSpec((NUM, H), lambda i: (0, 0)),
            pl.BlockSpec((1, H), lambda i: (0, 0)),
            pl.BlockSpec((1, H), lambda i: (0, 0)),
            pl.BlockSpec((1, 1), lambda i: (0, 0)),
        ],
        out_specs=pl.BlockSpec((BLK, 1), lambda i: (i, 0)),
        out_shape=jax.ShapeDtypeStruct((B, 1), jnp.float32),
    )(embs, num, w1a, w1b, b1r, w2r, b2r)


def kernel(categorical_inputs, numerical_inputs, tables, W1, b1, W2, b2):
    cat3 = categorical_inputs.astype(jnp.int32).reshape(NW, NCH, CH)
    off3 = jnp.asarray(_OFF_NP)
    tab2 = tables.reshape(F * V, D)
    embs5 = _sc_gather(cat3, off3, tab2)
    embs = embs5.reshape(B, F * D)
    w1a = W1[: F * D]
    w1b = W1[F * D :]
    return _mlp(
        embs,
        numerical_inputs,
        w1a,
        w1b,
        b1.reshape(1, H),
        W2.reshape(1, H),
        b2.reshape(1, 1),
    )


# speedup vs baseline: 8.0480x; 8.0480x over previous
import functools

import numpy as np
import jax
import jax.numpy as jnp
from jax import lax
from jax.experimental import pallas as pl
from jax.experimental.pallas import tpu as pltpu
from jax.experimental.pallas import tpu_sc as plsc

B = 16384
F = 26
V = 100000
D = 32
NUM = 13
H = 128

NW = 32
BPW = B // NW
IPW = BPW * F
CH = 128
NCH = IPW // CH
G = 4
NG = NCH // G

_OFF_NP = ((np.arange(B * F, dtype=np.int64) % F) * V).astype(np.int32)
_OFF_NP = _OFF_NP.reshape(NW, NCH, CH)


def _sc_gather(cat3, off3, tab2):
    mesh = plsc.VectorSubcoreMesh(core_axis_name="c", subcore_axis_name="s")

    @functools.partial(
        pl.kernel,
        mesh=mesh,
        compiler_params=pltpu.CompilerParams(use_tc_tiling_on_sc=False),
        out_type=jax.ShapeDtypeStruct((NW, NG, G, CH, D), jnp.float32),
        scratch_types=[
            pltpu.VMEM((NCH, CH), jnp.int32),
            pltpu.VMEM((NCH, CH), jnp.int32),
            pltpu.VMEM((G, CH, D), jnp.float32),
            pltpu.VMEM((G, CH, D), jnp.float32),
            pltpu.SemaphoreType.DMA,
            pltpu.SemaphoreType.DMA,
        ],
    )
    def k(cat_h, off_h, tab_h, out_h, idx_v, off_v, buf0, buf1, sem0, sem1):
        wid = lax.axis_index("s") * 2 + lax.axis_index("c")
        pltpu.sync_copy(cat_h.at[wid], idx_v)
        pltpu.sync_copy(off_h.at[wid], off_v)

        def add_body(i, carry):
            r = i // (CH // 16)
            c = (i % (CH // 16)) * 16
            idx_v[r, pl.ds(c, 16)] = (
                idx_v[r, pl.ds(c, 16)] + off_v[r, pl.ds(c, 16)]
            )
            return carry

        lax.fori_loop(0, NCH * (CH // 16), add_body, 0)

        def fire(g, buf, sem):
            return [
                pltpu.async_copy(tab_h.at[idx_v.at[g * G + j]], buf.at[j], sem)
                for j in range(G)
            ]

        def group_body(i, carry):
            g0 = i * 2
            g1 = g0 + 1
            cps0 = fire(g0, buf0, sem0)
            cps1 = fire(g1, buf1, sem1)
            for cp in cps0:
                cp.wait()
            pltpu.sync_copy(buf0, out_h.at[wid, g0])
            for cp in cps1:
                cp.wait()
            pltpu.sync_copy(buf1, out_h.at[wid, g1])
            return carry

        lax.fori_loop(0, NG // 2, group_body, 0)

    return k(cat3, off3, tab2)


def _mlp(embs, num, w1a, w1b, b1r, w2r, b2r):
    BLK = 1024

    def body(e_ref, n_ref, w1a_ref, w1b_ref, b1_ref, w2_ref, b2_ref, o_ref):
        x = jnp.dot(e_ref[...], w1a_ref[...], preferred_element_type=jnp.float32)
        x = x + jnp.dot(n_ref[...], w1b_ref[...], preferred_element_type=jnp.float32)
        x = jnp.maximum(x + b1_ref[...], 0.0)
        y = jnp.sum(x * w2_ref[...], axis=1, keepdims=True) + b2_ref[...]
        o_ref[...] = jax.nn.sigmoid(y)

    return pl.pallas_call(
        body,
        grid=(B // BLK,),
        in_specs=[
            pl.BlockSpec((BLK, F * D), lambda i: (i, 0)),
            pl.BlockSpec((BLK, NUM), lambda i: (i, 0)),
            pl.BlockSpec((F * D, H), lambda i: (0, 0)),
            pl.BlockSpec((NUM, H), lambda i: (0, 0)),
            pl.BlockSpec((1, H), lambda i: (0, 0)),
            pl.BlockSpec((1, H), lambda i: (0, 0)),
            pl.BlockSpec((1, 1), lambda i: (0, 0)),
        ],
        out_specs=pl.BlockSpec((BLK, 1), lambda i: (i, 0)),
        out_shape=jax.ShapeDtypeStruct((B, 1), jnp.float32),
    )(embs, num, w1a, w1b, b1r, w2r, b2r)


def kernel(categorical_inputs, numerical_inputs, tables, W1, b1, W2, b2):
    cat3 = categorical_inputs.astype(jnp.int32).reshape(NW, NCH, CH)
    off3 = jnp.asarray(_OFF_NP)
    tab2 = tables.reshape(F * V, D)
    embs5 = _sc_gather(cat3, off3, tab2)
    embs = embs5.reshape(B, F * D)
    w1a = W1[: F * D]
    w1b = W1[F * D :]
    return _mlp(
        embs,
        numerical_inputs,
        w1a,
        w1b,
        b1.reshape(1, H),
        W2.reshape(1, H),
        b2.reshape(1, 1),
    )


# speedup vs baseline: 8.0853x; 1.0046x over previous
import functools

import jax
import jax.numpy as jnp
from jax import lax
from jax.experimental import pallas as pl
from jax.experimental.pallas import tpu as pltpu
from jax.experimental.pallas import tpu_sc as plsc

B = 16384
F = 26
V = 100000
D = 32
NUM = 13
H = 128

NW = 32
BPW = B // NW
CH = 128
NCF = BPW // CH


def _sc_gather(cat4, tab3):
    mesh = plsc.VectorSubcoreMesh(core_axis_name="c", subcore_axis_name="s")

    @functools.partial(
        pl.kernel,
        mesh=mesh,
        compiler_params=pltpu.CompilerParams(use_tc_tiling_on_sc=False),
        out_type=jax.ShapeDtypeStruct((B // CH, CH, F, D), jnp.float32),
        scratch_types=[
            pltpu.VMEM((F, NCF, CH), jnp.int32),
            pltpu.VMEM((NCF, CH, D), jnp.float32),
            pltpu.VMEM((NCF, CH, D), jnp.float32),
            pltpu.SemaphoreType.DMA,
            pltpu.SemaphoreType.DMA,
        ],
    )
    def k(cat_h, tab_h, out_h, idx_v, buf0, buf1, sem0, sem1):
        wid = lax.axis_index("s") * 2 + lax.axis_index("c")
        base = wid * NCF
        pltpu.sync_copy(cat_h.at[wid], idx_v)

        def fire(f, buf, sem):
            return [
                pltpu.async_copy(
                    tab_h.at[f].at[idx_v.at[f, j]], buf.at[j], sem
                )
                for j in range(NCF)
            ]

        def field_body(i, carry):
            f0 = i * 2
            f1 = f0 + 1
            cps0 = fire(f0, buf0, sem0)
            cps1 = fire(f1, buf1, sem1)
            for cp in cps0:
                cp.wait()
            pltpu.sync_copy(buf0, out_h.at[pl.ds(base, NCF), :, f0])
            for cp in cps1:
                cp.wait()
            pltpu.sync_copy(buf1, out_h.at[pl.ds(base, NCF), :, f1])
            return carry

        lax.fori_loop(0, F // 2, field_body, 0)

    return k(cat4, tab3)


def _mlp(embs, num, w1a, w1b, b1r, w2r, b2r):
    BLK = 1024

    def body(e_ref, n_ref, w1a_ref, w1b_ref, b1_ref, w2_ref, b2_ref, o_ref):
        x = jnp.dot(e_ref[...], w1a_ref[...], preferred_element_type=jnp.float32)
        x = x + jnp.dot(n_ref[...], w1b_ref[...], preferred_element_type=jnp.float32)
        x = jnp.maximum(x + b1_ref[...], 0.0)
        y = jnp.sum(x * w2_ref[...], axis=1, keepdims=True) + b2_ref[...]
        o_ref[...] = jax.nn.sigmoid(y)

    return pl.pallas_call(
        body,
        grid=(B // BLK,),
        in_specs=[
            pl.BlockSpec((BLK, F * D), lambda i: (i, 0)),
            pl.BlockSpec((BLK, NUM), lambda i: (i, 0)),
            pl.BlockSpec((F * D, H), lambda i: (0, 0)),
            pl.BlockSpec((NUM, H), lambda i: (0, 0)),
            pl.BlockSpec((1, H), lambda i: (0, 0)),
            pl.BlockSpec((1, H), lambda i: (0, 0)),
            pl.BlockSpec((1, 1), lambda i: (0, 0)),
        ],
        out_specs=pl.BlockSpec((BLK, 1), lambda i: (i, 0)),
        out_shape=jax.ShapeDtypeStruct((B, 1), jnp.float32),
    )(embs, num, w1a, w1b, b1r, w2r, b2r)


def kernel(categorical_inputs, numerical_inputs, tables, W1, b1, W2, b2):
    cat = categorical_inputs.astype(jnp.int32)
    cat4 = cat.reshape(NW, BPW, F).transpose(0, 2, 1).reshape(NW, F, NCF, CH)
    embs3 = _sc_gather(cat4, tables)
    embs = embs3.reshape(B, F * D)
    w1a = W1[: F * D]
    w1b = W1[F * D :]
    return _mlp(
        embs,
        numerical_inputs,
        w1a,
        w1b,
        b1.reshape(1, H),
        W2.reshape(1, H),
        b2.reshape(1, 1),
    )


# speedup vs baseline: 13.9994x; 1.7315x over previous
import functools

import jax
import jax.numpy as jnp
from jax import lax
from jax.experimental import pallas as pl
from jax.experimental.pallas import tpu as pltpu
from jax.experimental.pallas import tpu_sc as plsc

B = 16384
F = 26
V = 100000
D = 32
NUM = 13
H = 128

A = 24960
NJ = 13
AJ = A // NJ
TV = V - 4 * A
PR = (NJ + 1) * AJ

NW = 32
BPW = B // NW
IPW = BPW * F
CH = 128
NCH = IPW // CH
G = 4
NG = NCH // G


def _transpose_tables(tabT):
    def body(t_ref, e_ref, o_ref):
        j = pl.program_id(1)
        for jj in range(NJ):

            @pl.when(j == jj)
            def _():
                acc = None
                for s in range(4):
                    part = lax.dot_general(
                        t_ref[0, :, s * A + jj * AJ : s * A + (jj + 1) * AJ],
                        e_ref[s * D : (s + 1) * D, :],
                        (((0,), (0,)), ((), ())),
                        preferred_element_type=jnp.float32,
                    )
                    acc = part if acc is None else acc + part
                o_ref[...] = acc

        @pl.when(j == NJ)
        def _():
            tail = lax.dot_general(
                t_ref[0, :, 4 * A : V],
                e_ref[0:D, :],
                (((0,), (0,)), ((), ())),
                preferred_element_type=jnp.float32,
            )
            o_ref[0:TV, :] = tail

    return pl.pallas_call(
        body,
        grid=(F, NJ + 1),
        in_specs=[
            pl.BlockSpec((1, D, V), lambda f, j: (f, 0, 0)),
            pl.BlockSpec((4 * D, 4 * D), lambda f, j: (0, 0)),
        ],
        out_specs=pl.BlockSpec((AJ, 4 * D), lambda f, j: (f * (NJ + 1) + j, 0)),
        out_shape=jax.ShapeDtypeStruct((F * (NJ + 1) * AJ, 4 * D), jnp.float32),
    )(tabT, jnp.eye(4 * D, dtype=jnp.float32))


def _sc_gather(cat3, tab2):
    mesh = plsc.VectorSubcoreMesh(core_axis_name="c", subcore_axis_name="s")

    @functools.partial(
        pl.kernel,
        mesh=mesh,
        compiler_params=pltpu.CompilerParams(use_tc_tiling_on_sc=False),
        out_type=jax.ShapeDtypeStruct((NW, NG, G, CH, D), jnp.float32),
        scratch_types=[
            pltpu.VMEM((NCH, CH), jnp.int32),
            pltpu.VMEM((G, CH, D), jnp.float32),
            pltpu.VMEM((G, CH, D), jnp.float32),
            pltpu.SemaphoreType.DMA,
            pltpu.SemaphoreType.DMA,
        ],
    )
    def k(cat_h, tab_h, out_h, idx_v, buf0, buf1, sem0, sem1):
        wid = lax.axis_index("s") * 2 + lax.axis_index("c")
        pltpu.sync_copy(cat_h.at[wid], idx_v)

        def fire(g, buf, sem):
            return [
                pltpu.async_copy(tab_h.at[idx_v.at[g * G + j]], buf.at[j], sem)
                for j in range(G)
            ]

        def group_body(i, carry):
            g0 = i * 2
            g1 = g0 + 1
            cps0 = fire(g0, buf0, sem0)
            cps1 = fire(g1, buf1, sem1)
            for cp in cps0:
                cp.wait()
            pltpu.sync_copy(buf0, out_h.at[wid, g0])
            for cp in cps1:
                cp.wait()
            pltpu.sync_copy(buf1, out_h.at[wid, g1])
            return carry

        lax.fori_loop(0, NG // 2, group_body, 0)

    return k(cat3, tab2)


def _mlp(embs, num, w1a, w1b, b1r, w2r, b2r):
    BLK = 1024

    def body(e_ref, n_ref, w1a_ref, w1b_ref, b1_ref, w2_ref, b2_ref, o_ref):
        x = jnp.dot(e_ref[...], w1a_ref[...], preferred_element_type=jnp.float32)
        x = x + jnp.dot(n_ref[...], w1b_ref[...], preferred_element_type=jnp.float32)
        x = jnp.maximum(x + b1_ref[...], 0.0)
        y = jnp.sum(x * w2_ref[...], axis=1, keepdims=True) + b2_ref[...]
        o_ref[...] = jax.nn.sigmoid(y)

    return pl.pallas_call(
        body,
        grid=(B // BLK,),
        in_specs=[
            pl.BlockSpec((BLK, F * D), lambda i: (i, 0)),
            pl.BlockSpec((BLK, NUM), lambda i: (i, 0)),
            pl.BlockSpec((F * D, H), lambda i: (0, 0)),
            pl.BlockSpec((NUM, H), lambda i: (0, 0)),
            pl.BlockSpec((1, H), lambda i: (0, 0)),
            pl.BlockSpec((1, H), lambda i: (0, 0)),
            pl.BlockSpec((1, 1), lambda i: (0, 0)),
        ],
        out_specs=pl.BlockSpec((BLK, 1), lambda i: (i, 0)),
        out_shape=jax.ShapeDtypeStruct((B, 1), jnp.float32),
    )(embs, num, w1a, w1b, b1r, w2r, b2r)


def kernel(categorical_inputs, numerical_inputs, tables, W1, b1, W2, b2):
    tabT = tables.transpose(0, 2, 1)
    tabP = _transpose_tables(tabT)
    tab2 = tabP.reshape(F * PR * 4, D)
    v = categorical_inputs.astype(jnp.int32)
    s = v // A
    offs = jnp.arange(F, dtype=jnp.int32) * PR
    idx = jnp.where(
        s < 4,
        4 * (offs[None, :] + v - s * A) + s,
        4 * (offs[None, :] + v - 3 * A),
    )
    cat3 = idx.reshape(NW, NCH, CH)
    embs5 = _sc_gather(cat3, tab2)
    embs = embs5.reshape(B, F * D)
    w1a = W1[: F * D]
    w1b = W1[F * D :]
    return _mlp(
        embs,
        numerical_inputs,
        w1a,
        w1b,
        b1.reshape(1, H),
        W2.reshape(1, H),
        b2.reshape(1, 1),
    )


# speedup vs baseline: 22.1725x; 1.5838x over previous
import functools

import jax
import jax.numpy as jnp
from jax import lax
from jax.experimental import pallas as pl
from jax.experimental.pallas import tpu as pltpu
from jax.experimental.pallas import tpu_sc as plsc

B = 16384
F = 26
V = 100000
D = 32
NUM = 13
H = 128

A = 24960
NJ = 5
AJ = A // NJ
TV = V - 4 * A
PR = (NJ + 1) * AJ

NW = 32
BPW = B // NW
IPW = BPW * F
CH = 128
NCH = IPW // CH
G = 4
NG = NCH // G


def _transpose_tables(tabT):
    def body(t_ref, e_ref, o_ref):
        j = pl.program_id(1)
        for jj in range(NJ):

            @pl.when(j == jj)
            def _():
                stacked = jnp.concatenate(
                    [
                        t_ref[0, :, s * A + jj * AJ : s * A + (jj + 1) * AJ]
                        for s in range(4)
                    ],
                    axis=0,
                )
                o_ref[...] = lax.dot_general(
                    stacked,
                    e_ref[...],
                    (((0,), (0,)), ((), ())),
                    preferred_element_type=jnp.float32,
                )

        @pl.when(j == NJ)
        def _():
            tail = lax.dot_general(
                t_ref[0, :, 4 * A : V],
                e_ref[0:D, :],
                (((0,), (0,)), ((), ())),
                preferred_element_type=jnp.float32,
            )
            o_ref[0:TV, :] = tail

    return pl.pallas_call(
        body,
        grid=(F, NJ + 1),
        in_specs=[
            pl.BlockSpec((1, D, V), lambda f, j: (f, 0, 0)),
            pl.BlockSpec((4 * D, 4 * D), lambda f, j: (0, 0)),
        ],
        out_specs=pl.BlockSpec((AJ, 4 * D), lambda f, j: (f * (NJ + 1) + j, 0)),
        out_shape=jax.ShapeDtypeStruct((F * (NJ + 1) * AJ, 4 * D), jnp.float32),
    )(tabT, jnp.eye(4 * D, dtype=jnp.float32))


def _sc_gather(cat3, tab2):
    mesh = plsc.VectorSubcoreMesh(core_axis_name="c", subcore_axis_name="s")

    @functools.partial(
        pl.kernel,
        mesh=mesh,
        compiler_params=pltpu.CompilerParams(use_tc_tiling_on_sc=False),
        out_type=jax.ShapeDtypeStruct((NW, NG, G, CH, D), jnp.float32),
        scratch_types=[
            pltpu.VMEM((NCH, CH), jnp.int32),
            pltpu.VMEM((G, CH, D), jnp.float32),
            pltpu.VMEM((G, CH, D), jnp.float32),
            pltpu.SemaphoreType.DMA,
            pltpu.SemaphoreType.DMA,
        ],
    )
    def k(cat_h, tab_h, out_h, idx_v, buf0, buf1, sem0, sem1):
        wid = lax.axis_index("s") * 2 + lax.axis_index("c")
        pltpu.sync_copy(cat_h.at[wid], idx_v)

        def fire(g, buf, sem):
            return [
                pltpu.async_copy(tab_h.at[idx_v.at[g * G + j]], buf.at[j], sem)
                for j in range(G)
            ]

        def group_body(i, carry):
            g0 = i * 2
            g1 = g0 + 1
            cps0 = fire(g0, buf0, sem0)
            cps1 = fire(g1, buf1, sem1)
            for cp in cps0:
                cp.wait()
            pltpu.sync_copy(buf0, out_h.at[wid, g0])
            for cp in cps1:
                cp.wait()
            pltpu.sync_copy(buf1, out_h.at[wid, g1])
            return carry

        lax.fori_loop(0, NG // 2, group_body, 0)

    return k(cat3, tab2)


def _mlp(embs, num, w1a, w1b, b1r, w2r, b2r):
    BLK = 1024

    def body(e_ref, n_ref, w1a_ref, w1b_ref, b1_ref, w2_ref, b2_ref, o_ref):
        x = jnp.dot(e_ref[...], w1a_ref[...], preferred_element_type=jnp.float32)
        x = x + jnp.dot(n_ref[...], w1b_ref[...], preferred_element_type=jnp.float32)
        x = jnp.maximum(x + b1_ref[...], 0.0)
        y = jnp.sum(x * w2_ref[...], axis=1, keepdims=True) + b2_ref[...]
        o_ref[...] = jax.nn.sigmoid(y)

    return pl.pallas_call(
        body,
        grid=(B // BLK,),
        in_specs=[
            pl.BlockSpec((BLK, F * D), lambda i: (i, 0)),
            pl.BlockSpec((BLK, NUM), lambda i: (i, 0)),
            pl.BlockSpec((F * D, H), lambda i: (0, 0)),
            pl.BlockSpec((NUM, H), lambda i: (0, 0)),
            pl.BlockSpec((1, H), lambda i: (0, 0)),
            pl.BlockSpec((1, H), lambda i: (0, 0)),
            pl.BlockSpec((1, 1), lambda i: (0, 0)),
        ],
        out_specs=pl.BlockSpec((BLK, 1), lambda i: (i, 0)),
        out_shape=jax.ShapeDtypeStruct((B, 1), jnp.float32),
    )(embs, num, w1a, w1b, b1r, w2r, b2r)


def kernel(categorical_inputs, numerical_inputs, tables, W1, b1, W2, b2):
    tabT = tables.transpose(0, 2, 1)
    tabP = _transpose_tables(tabT)
    tab2 = tabP.reshape(F * PR * 4, D)
    v = categorical_inputs.astype(jnp.int32)
    s = v // A
    offs = jnp.arange(F, dtype=jnp.int32) * PR
    idx = jnp.where(
        s < 4,
        4 * (offs[None, :] + v - s * A) + s,
        4 * (offs[None, :] + v - 3 * A),
    )
    cat3 = idx.reshape(NW, NCH, CH)
    embs5 = _sc_gather(cat3, tab2)
    embs = embs5.reshape(B, F * D)
    w1a = W1[: F * D]
    w1b = W1[F * D :]
    return _mlp(
        embs,
        numerical_inputs,
        w1a,
        w1b,
        b1.reshape(1, H),
        W2.reshape(1, H),
        b2.reshape(1, 1),
    )
